# Initial kernel scaffold; baseline (speedup 1.0000x reference)
import jax
import jax.numpy as jnp
from jax.experimental import pallas as pl


def kernel(x, edge_index, edge_vec, edge_len, w_proj, radial_w, tangential_w, w_out, radial_score, tangential_score, log_scale, temp_bias, temp_weight):
    raise NotImplementedError("write your pallas kernel here")


# speedup vs baseline: 9.2726x; 9.2726x over previous
import functools
import jax
import jax.numpy as jnp
from jax import lax
from jax.experimental import pallas as pl
from jax.experimental.pallas import tpu as pltpu


_NB = 1000


def _k1_body(x_ref, w_ref, wproj_ref, rs_ref, ts_ref, p_ref, es_ref):
    xb = x_ref[...]
    k2 = w_ref.shape[0]
    h = k2 // 2
    cols = []
    for i in range(h):
        wp = wproj_ref[i]
        cols.append(jnp.dot(wp, rs_ref[i][:, None]))
    for i in range(h):
        wp = wproj_ref[i]
        cols.append(jnp.dot(wp, ts_ref[i][:, None]))
    vmat = jnp.concatenate(cols, axis=1)
    es_ref[...] = jnp.dot(xb, vmat)
    for k in range(k2):
        p_ref[:, k, :] = jnp.dot(xb, w_ref[k])


def _k1(x, w_stack, w_proj, r_score, t_score):
    n, f = x.shape
    k2 = w_stack.shape[0]
    grid = n // _NB
    return pl.pallas_call(
        _k1_body,
        grid=(grid,),
        in_specs=[
            pl.BlockSpec((_NB, f), lambda i: (i, 0)),
            pl.BlockSpec((k2, f, f), lambda i: (0, 0, 0)),
            pl.BlockSpec((k2 // 2, f, f), lambda i: (0, 0, 0)),
            pl.BlockSpec((k2 // 2, f), lambda i: (0, 0)),
            pl.BlockSpec((k2 // 2, f), lambda i: (0, 0)),
        ],
        out_specs=[
            pl.BlockSpec((_NB, k2, f), lambda i: (i, 0, 0)),
            pl.BlockSpec((_NB, k2), lambda i: (i, 0)),
        ],
        out_shape=[
            jax.ShapeDtypeStruct((n, k2, f), jnp.float32),
            jax.ShapeDtypeStruct((n, k2), jnp.float32),
        ],
    )(x, w_stack, w_proj, r_score, t_score)


def _k5_body(x_ref, msg_ref, sa_ref, p_ref, wout_ref, o_ref, *, nheads):
    xb = x_ref[...]
    acc = msg_ref[...]
    k2 = p_ref.shape[1]
    for k in range(k2):
        acc = acc - sa_ref[:, k][:, None] * p_ref[:, k, :]
    acc = acc * (1.0 / nheads)
    acc = jnp.nan_to_num(acc)
    o_ref[...] = xb + jnp.dot(acc, wout_ref[...])


def _k5(x, msg, sa, p, w_out, nheads):
    n, f = x.shape
    k2 = p.shape[1]
    grid = n // _NB
    return pl.pallas_call(
        functools.partial(_k5_body, nheads=nheads),
        grid=(grid,),
        in_specs=[
            pl.BlockSpec((_NB, f), lambda i: (i, 0)),
            pl.BlockSpec((_NB, f), lambda i: (i, 0)),
            pl.BlockSpec((_NB, k2), lambda i: (i, 0)),
            pl.BlockSpec((_NB, k2, f), lambda i: (i, 0, 0)),
            pl.BlockSpec((f, f), lambda i: (0, 0)),
        ],
        out_specs=pl.BlockSpec((_NB, f), lambda i: (i, 0)),
        out_shape=jax.ShapeDtypeStruct((n, f), jnp.float32),
    )(x, msg, sa, p, w_out)


def kernel(x, edge_index, edge_vec, edge_len, w_proj, radial_w, tangential_w,
           w_out, radial_score, tangential_score, log_scale, temp_bias,
           temp_weight):
    n, f = x.shape
    h = w_proj.shape[0]
    k2 = 2 * h
    s = edge_index[0]
    r = edge_index[1]

    w_stack = jnp.concatenate([radial_w, tangential_w], axis=0)
    p, es = _k1(x, w_stack, w_proj, radial_score, tangential_score)

    scale = jax.nn.softplus(log_scale)
    d = es[s] - es[r]
    temp = jax.nn.softplus(temp_bias[None, :] + temp_weight[None, :] * edge_len[:, None]) + 1e-4
    logit_r = (d[:, :h] - scale * edge_len[:, None]) / temp
    logits = jnp.concatenate([logit_r, d[:, h:]], axis=1)
    m = jnp.max(logits, axis=0)
    ex = jnp.exp(logits - m[None, :])
    den = jax.ops.segment_sum(ex, r, num_segments=n)
    alpha = ex / den[r]
    msg = jax.ops.segment_sum(
        jnp.einsum('ek,ekg->eg', alpha, p[s]), r, num_segments=n)
    sa = jax.ops.segment_sum(alpha, r, num_segments=n)

    return _k5(x, msg, sa, p, w_out, h)


# speedup vs baseline: 18.2669x; 1.9700x over previous
import functools
import jax
import jax.numpy as jnp
from jax import lax
from jax.experimental import pallas as pl
from jax.experimental.pallas import tpu as pltpu
from jax.experimental.pallas import tpu_sc as plsc

N = 10000
F = 128
H = 4
K2 = 2 * H
E = 320000
L = 16
NC = 2
NS = 16
EP = 327680
EROWS = EP // 128
RPW = EROWS // NS
NT = 10112
NRW = NT // NS
G1 = 5
G2 = 10

_NB = 1000


def _k1_body(x_ref, w_ref, wproj_ref, rs_ref, ts_ref, p_ref, es_ref):
    xb = x_ref[...]
    cols = []
    for i in range(H):
        cols.append(jnp.dot(wproj_ref[i], rs_ref[i][:, None]))
    for i in range(H):
        cols.append(jnp.dot(wproj_ref[i], ts_ref[i][:, None]))
    vmat = jnp.concatenate(cols, axis=1)
    es_ref[...] = jnp.dot(xb, vmat)
    for k in range(K2):
        p_ref[k] = jnp.dot(xb, w_ref[k])


def _k1(x, w_stack, w_proj, r_score, t_score):
    grid = N // _NB
    return pl.pallas_call(
        _k1_body,
        grid=(grid,),
        in_specs=[
            pl.BlockSpec((_NB, F), lambda i: (i, 0)),
            pl.BlockSpec((K2, F, F), lambda i: (0, 0, 0)),
            pl.BlockSpec((H, F, F), lambda i: (0, 0, 0)),
            pl.BlockSpec((H, F), lambda i: (0, 0)),
            pl.BlockSpec((H, F), lambda i: (0, 0)),
        ],
        out_specs=[
            pl.BlockSpec((K2, _NB, F), lambda i: (0, i, 0)),
            pl.BlockSpec((_NB, K2), lambda i: (i, 0)),
        ],
        out_shape=[
            jax.ShapeDtypeStruct((K2, N, F), jnp.float32),
            jax.ShapeDtypeStruct((N, K2), jnp.float32),
        ],
    )(x, w_stack, w_proj, r_score, t_score)


def _k1b_body(len_ref, params_ref, ls_ref, a_ref, b_ref):
    lb = len_ref[...]
    scale = jax.nn.softplus(ls_ref[0])
    for k in range(H):
        t = jax.nn.softplus(params_ref[0, k] + params_ref[1, k] * lb)
        a = 1.0 / (t + 1e-4)
        a_ref[k] = a
        b_ref[k] = -scale * lb * a


def _k1b(edge_len, temp_bias, temp_weight, log_scale):
    len2 = edge_len.reshape(E // 128, 128)
    params = jnp.stack([temp_bias, temp_weight])
    ls = log_scale.reshape(1)
    return pl.pallas_call(
        _k1b_body,
        grid=(1,),
        in_specs=[
            pl.BlockSpec((E // 128, 128), lambda i: (0, 0)),
            pl.BlockSpec(memory_space=pltpu.SMEM),
            pl.BlockSpec(memory_space=pltpu.SMEM),
        ],
        out_specs=[
            pl.BlockSpec((H, E // 128, 128), lambda i: (0, 0, 0)),
            pl.BlockSpec((H, E // 128, 128), lambda i: (0, 0, 0)),
        ],
        out_shape=[
            jax.ShapeDtypeStruct((H, E // 128, 128), jnp.float32),
            jax.ShapeDtypeStruct((H, E // 128, 128), jnp.float32),
        ],
    )(len2, params, ls)


def _iota16():
    return lax.iota(jnp.int32, L)


FH = F // 2
DW = 8


def _k234_body(es_h, sidx2, ridx2, av_h, bv_h, pf0, pf1, zmsg, zden,
               msgp, denp, lscr,
               sidx_v, ridx_v, esS, esR, av, bv, lb, exb, mxv, maxrd,
               gidx, didx, pg, combo, db, alb, sem,
               msg_sh, den_sh, maxsh):
    c = lax.axis_index("c")
    s = lax.axis_index("s")
    is_sc0 = c == 0
    row0 = s * RPW
    nr0 = s * NRW

    pltpu.sync_copy(zmsg.at[pl.ds(nr0, NRW)], msg_sh.at[pl.ds(nr0, NRW)])
    pltpu.sync_copy(zden.at[pl.ds(nr0, NRW)], den_sh.at[pl.ds(nr0, NRW)])

    def p1_body(j, mxs):
        rb = row0 + j * G1
        pltpu.sync_copy(sidx2.at[pl.ds(rb, G1)], sidx_v.at[pl.ds(0, G1)])
        pltpu.sync_copy(ridx2.at[pl.ds(rb, G1)], ridx_v.at[pl.ds(0, G1)])
        pltpu.sync_copy(av_h.at[:, pl.ds(rb, G1), :], av)
        pltpu.sync_copy(bv_h.at[:, pl.ds(rb, G1), :], bv)
        hs = []
        for g in range(G1):
            hs.append(pltpu.async_copy(es_h.at[sidx_v.at[g]],
                                       esS.at[pl.ds(g * 128, 128)], sem))
            hs.append(pltpu.async_copy(es_h.at[ridx_v.at[g]],
                                       esR.at[pl.ds(g * 128, 128)], sem))
        for hh in hs:
            hh.wait()
        def p1_row(r2, mxs_in):
            new_mxs = list(mxs_in)
            for kk in range(H):
                kglob = c * H + kk
                cols = jnp.full((L,), kglob, jnp.int32)
                m = new_mxs[kk]
                for sub in range(128 // L):
                    rows = _iota16() + (r2 * 128 + sub * L)
                    vs = plsc.load_gather(esS, [rows, cols])
                    vr = plsc.load_gather(esR, [rows, cols])
                    d = vs - vr
                    off = sub * L
                    a = av[kk, r2, off:off + L]
                    b = bv[kk, r2, off:off + L]
                    lval = jnp.where(is_sc0, d * a + b, d)
                    lb[kk, r2, off:off + L] = lval
                    m = jnp.maximum(m, lval)
                new_mxs[kk] = m
            return tuple(new_mxs)

        new_mxs = lax.fori_loop(0, G1, p1_row, mxs)
        pltpu.sync_copy(lb.at[:, pl.ds(0, G1), :],
                        lscr.at[c, :, pl.ds(rb, G1), :])
        return new_mxs

    mxs0 = tuple(jnp.full((L,), -3.0e38, jnp.float32) for _ in range(H))
    mxs = lax.fori_loop(0, RPW // G1, p1_body, mxs0)
    for kk in range(H):
        mxv[kk] = mxs[kk]
    pltpu.sync_copy(mxv, maxsh.at[s])
    plsc.subcore_barrier()

    pltpu.sync_copy(maxsh, maxrd)
    M = []
    for kk in range(H):
        acc = maxrd[0, kk]
        for i in range(1, NS):
            acc = jnp.maximum(acc, maxrd[i, kk])
        M.append(jnp.max(acc))

    def p2_body(j, carry):
        rb = row0 + j * G2
        pltpu.sync_copy(ridx2.at[pl.ds(rb, G2)], ridx_v)
        pltpu.sync_copy(lscr.at[c, :, pl.ds(rb, G2), :], lb)

        def p2_row(r2, cc):
            for kk in range(H):
                cols = jnp.full((L,), kk, jnp.int32)
                for sub in range(128 // L):
                    off = sub * L
                    ex = jnp.exp(lb[kk, r2, off:off + L] - M[kk])
                    plsc.store_scatter(
                        exb, [_iota16() + (r2 * 128 + off), cols], ex)
            pltpu.sync_copy(exb.at[pl.ds(r2 * 128, 128)],
                            den_sh.at[ridx_v.at[r2]], add=True)
            return cc

        lax.fori_loop(0, G2, p2_row, 0)
        return carry

    lax.fori_loop(0, RPW // G2, p2_body, 0)
    plsc.subcore_barrier()
    pltpu.sync_copy(den_sh.at[pl.ds(nr0, NRW)],
                    denp.at[pl.dslice(c * NT + nr0, NRW)])
    plsc.subcore_barrier()

    def p3_pass(pf):
        def p3_body(r, carry):
            rb = row0 + r
            pltpu.sync_copy(sidx2.at[pl.ds(rb, 1)], sidx_v.at[pl.ds(0, 1)])
            pltpu.sync_copy(ridx2.at[pl.ds(rb, 1)], ridx_v.at[pl.ds(0, 1)])
            pltpu.sync_copy(lscr.at[c, :, pl.ds(rb, 1), :],
                            lb.at[:, pl.ds(0, 1), :])
            for g in range(128 // L):
                vr = ridx_v[0, g * L:(g + 1) * L]
                didx[g * L:(g + 1) * L] = vr + c * NT
            hden = pltpu.async_copy(denp.at[didx], db, sem)
            for kk in range(H):
                koff = (c * H + kk) * N
                for g in range(128 // L):
                    v = sidx_v[0, g * L:(g + 1) * L]
                    gidx[kk, g * L:(g + 1) * L] = v + koff
            hs = [pltpu.async_copy(pf.at[gidx.at[kk]], pg.at[kk], sem)
                  for kk in range(H)]
            hden.wait()
            for kk in range(H):
                cols = jnp.full((L,), kk, jnp.int32)
                for g in range(128 // L):
                    lv = lb[kk, 0, g * L:(g + 1) * L]
                    ex = jnp.exp(lv - M[kk])
                    dv = plsc.load_gather(db, [_iota16() + g * L, cols])
                    alb[kk, g * L:(g + 1) * L] = ex / dv
            for hh in hs:
                hh.wait()

            def grp_body(gi, cc):
                gsl = pl.ds(gi * L, L)
                va = [alb[kk, gsl] for kk in range(H)]
                for li in range(L):
                    row = gi * L + li
                    a0 = va[0][li]
                    a1 = va[1][li]
                    a2 = va[2][li]
                    a3 = va[3][li]
                    for fb in range(FH // L):
                        sl = pl.ds(fb * L, L)
                        acc = (a0 * pg[0, row, sl] + a1 * pg[1, row, sl]
                               + a2 * pg[2, row, sl] + a3 * pg[3, row, sl])
                        combo[row, sl] = acc
                return cc

            lax.fori_loop(0, 128 // L, grp_body, 0)
            pltpu.sync_copy(combo, msg_sh.at[ridx_v.at[0]], add=True)
            return carry

        lax.fori_loop(0, RPW, p3_body, 0)

    p3_pass(pf0)
    plsc.subcore_barrier()
    pltpu.sync_copy(msg_sh.at[pl.ds(nr0, NRW)], msgp.at[c, 0, pl.ds(nr0, NRW)])
    pltpu.sync_copy(zmsg.at[pl.ds(nr0, NRW)], msg_sh.at[pl.ds(nr0, NRW)])
    plsc.subcore_barrier()
    p3_pass(pf1)
    plsc.subcore_barrier()

    pltpu.sync_copy(msg_sh.at[pl.ds(nr0, NRW)], msgp.at[c, 1, pl.ds(nr0, NRW)])


def _k234(es_p, sidx2, ridx2, a2, b2, pf0, pf1, zmsg, zden):
    mesh = plsc.VectorSubcoreMesh(core_axis_name="c", subcore_axis_name="s")
    fn = functools.partial(
        pl.kernel,
        out_type=[
            jax.ShapeDtypeStruct((NC, 2, NT, FH), jnp.float32),
            jax.ShapeDtypeStruct((NC * NT, DW), jnp.float32),
            jax.ShapeDtypeStruct((NC, H, EROWS, 128), jnp.float32),
        ],
        mesh=mesh,
        compiler_params=pltpu.CompilerParams(use_tc_tiling_on_sc=False,
                                             needs_layout_passes=False),
        scratch_types=[
            pltpu.VMEM((G2, 128), jnp.int32),
            pltpu.VMEM((G2, 128), jnp.int32),
            pltpu.VMEM((G1 * 128, K2), jnp.float32),
            pltpu.VMEM((G1 * 128, K2), jnp.float32),
            pltpu.VMEM((H, G1, 128), jnp.float32),
            pltpu.VMEM((H, G1, 128), jnp.float32),
            pltpu.VMEM((H, G2, 128), jnp.float32),
            pltpu.VMEM((G2 * 128, DW), jnp.float32),
            pltpu.VMEM((H, L), jnp.float32),
            pltpu.VMEM((NS, H, L), jnp.float32),
            pltpu.VMEM((H, 128), jnp.int32),
            pltpu.VMEM((128,), jnp.int32),
            pltpu.VMEM((H, 128, FH), jnp.float32),
            pltpu.VMEM((128, FH), jnp.float32),
            pltpu.VMEM((128, DW), jnp.float32),
            pltpu.VMEM((H, 128), jnp.float32),
            pltpu.SemaphoreType.DMA,
            pltpu.VMEM_SHARED((NT, FH), jnp.float32),
            pltpu.VMEM_SHARED((NT, DW), jnp.float32),
            pltpu.VMEM_SHARED((NS, H, L), jnp.float32),
        ],
    )
    return fn(_k234_body)(es_p, sidx2, ridx2, a2, b2, pf0, pf1, zmsg, zden)


def _k5_body(x_ref, msg_ref, den_ref, p_ref, wout_ref, o_ref):
    xb = x_ref[...]
    acc = jnp.concatenate(
        [msg_ref[0, 0] + msg_ref[1, 0], msg_ref[0, 1] + msg_ref[1, 1]],
        axis=-1)
    for k in range(K2):
        dcol = den_ref[k // H][:, k % H]
        ind = jnp.where(dcol > 0.0, 1.0, 0.0)
        acc = acc - ind[:, None] * p_ref[k]
    acc = acc * (1.0 / H)
    acc = jnp.nan_to_num(acc)
    o_ref[...] = xb + jnp.dot(acc, wout_ref[...])


def _k5(x, msgp, denp, p, w_out):
    grid = N // _NB
    return pl.pallas_call(
        _k5_body,
        grid=(grid,),
        in_specs=[
            pl.BlockSpec((_NB, F), lambda i: (i, 0)),
            pl.BlockSpec((NC, 2, _NB, FH), lambda i: (0, 0, i, 0)),
            pl.BlockSpec((NC, _NB, DW), lambda i: (0, i, 0)),
            pl.BlockSpec((K2, _NB, F), lambda i: (0, i, 0)),
            pl.BlockSpec((F, F), lambda i: (0, 0)),
        ],
        out_specs=pl.BlockSpec((_NB, F), lambda i: (i, 0)),
        out_shape=jax.ShapeDtypeStruct((N, F), jnp.float32),
    )(x, msgp, denp, p, w_out)


def kernel(x, edge_index, edge_vec, edge_len, w_proj, radial_w, tangential_w,
           w_out, radial_score, tangential_score, log_scale, temp_bias,
           temp_weight):
    w_stack = jnp.concatenate([radial_w, tangential_w], axis=0)
    p, es = _k1(x, w_stack, w_proj, radial_score, tangential_score)
    a_e, b_e = _k1b(edge_len, temp_bias, temp_weight, log_scale)

    es_p = jnp.pad(es, ((0, NT - N), (0, 0)))
    sidx2 = jnp.pad(edge_index[0], (0, EP - E)).reshape(EROWS, 128)
    ridx2 = jnp.pad(edge_index[1], (0, EP - E),
                    constant_values=N).reshape(EROWS, 128)
    a2 = jnp.pad(a_e.reshape(H, E), ((0, 0), (0, EP - E))).reshape(
        H, EROWS, 128)
    b2 = jnp.pad(b_e.reshape(H, E), ((0, 0), (0, EP - E))).reshape(
        H, EROWS, 128)
    pflat = p.reshape(K2 * N, F)
    pf0 = pflat[:, :FH]
    pf1 = pflat[:, FH:]
    zmsg = jnp.zeros((NT, FH), jnp.float32)
    zden = jnp.zeros((NT, DW), jnp.float32)

    msgp, denp, _ = _k234(es_p, sidx2, ridx2, a2, b2, pf0, pf1, zmsg, zden)
    return _k5(x, msgp, denp.reshape(NC, NT, DW), p, w_out)


# speedup vs baseline: 19.3202x; 1.0577x over previous
import functools
import jax
import jax.numpy as jnp
from jax import lax
from jax.experimental import pallas as pl
from jax.experimental.pallas import tpu as pltpu
from jax.experimental.pallas import tpu_sc as plsc

N = 10000
F = 128
H = 4
K2 = 2 * H
E = 320000
L = 16
NC = 2
NS = 16
EP = 327680
EROWS = EP // 128
RPW = EROWS // NS
NT = 10112
NRW = NT // NS
G1 = 4
G2 = 5
KB = 8

_NB = 1000


def _k1_body(x_ref, w_ref, wproj_ref, rs_ref, ts_ref, p_ref, es_ref):
    xb = x_ref[...]
    cols = []
    for i in range(H):
        cols.append(jnp.dot(wproj_ref[i], rs_ref[i][:, None]))
    for i in range(H):
        cols.append(jnp.dot(wproj_ref[i], ts_ref[i][:, None]))
    vmat = jnp.concatenate(cols, axis=1)
    es_ref[...] = jnp.dot(xb, vmat)
    for k in range(K2):
        p_ref[k] = jnp.dot(xb, w_ref[k])


def _k1(x, w_stack, w_proj, r_score, t_score):
    grid = N // _NB
    return pl.pallas_call(
        _k1_body,
        grid=(grid,),
        in_specs=[
            pl.BlockSpec((_NB, F), lambda i: (i, 0)),
            pl.BlockSpec((K2, F, F), lambda i: (0, 0, 0)),
            pl.BlockSpec((H, F, F), lambda i: (0, 0, 0)),
            pl.BlockSpec((H, F), lambda i: (0, 0)),
            pl.BlockSpec((H, F), lambda i: (0, 0)),
        ],
        out_specs=[
            pl.BlockSpec((K2, _NB, F), lambda i: (0, i, 0)),
            pl.BlockSpec((_NB, K2), lambda i: (i, 0)),
        ],
        out_shape=[
            jax.ShapeDtypeStruct((K2, N, F), jnp.float32),
            jax.ShapeDtypeStruct((N, K2), jnp.float32),
        ],
    )(x, w_stack, w_proj, r_score, t_score)


def _k1b_body(len_ref, params_ref, ls_ref, a_ref, b_ref):
    lb = len_ref[...]
    scale = jax.nn.softplus(ls_ref[0])
    for k in range(H):
        t = jax.nn.softplus(params_ref[0, k] + params_ref[1, k] * lb)
        a = 1.0 / (t + 1e-4)
        a_ref[k] = a
        b_ref[k] = -scale * lb * a


def _k1b(edge_len, temp_bias, temp_weight, log_scale):
    len2 = edge_len.reshape(E // 128, 128)
    params = jnp.stack([temp_bias, temp_weight])
    ls = log_scale.reshape(1)
    return pl.pallas_call(
        _k1b_body,
        grid=(1,),
        in_specs=[
            pl.BlockSpec((E // 128, 128), lambda i: (0, 0)),
            pl.BlockSpec(memory_space=pltpu.SMEM),
            pl.BlockSpec(memory_space=pltpu.SMEM),
        ],
        out_specs=[
            pl.BlockSpec((H, E // 128, 128), lambda i: (0, 0, 0)),
            pl.BlockSpec((H, E // 128, 128), lambda i: (0, 0, 0)),
        ],
        out_shape=[
            jax.ShapeDtypeStruct((H, E // 128, 128), jnp.float32),
            jax.ShapeDtypeStruct((H, E // 128, 128), jnp.float32),
        ],
    )(len2, params, ls)


def _iota16():
    return lax.iota(jnp.int32, L)


FH = F // 2
DW = 8


def _k234_body(es_h, sidx2, ridx2, av_h, bv_h, pf0, pf1, zmsg, zden,
               msgp, denp, lscr,
               sidx_v, ridx_v, esS, esR, av, bv, lb, exb, mxv, maxrd,
               gidx, didx, pg, combo, db, alb, sem,
               msg_sh, den_sh, maxsh):
    c = lax.axis_index("c")
    s = lax.axis_index("s")
    is_sc0 = c == 0
    row0 = s * RPW
    nr0 = s * NRW

    pltpu.sync_copy(zmsg.at[pl.ds(nr0, NRW)], msg_sh.at[pl.ds(nr0, NRW)])
    pltpu.sync_copy(zden.at[pl.ds(nr0, NRW)], den_sh.at[pl.ds(nr0, NRW)])

    def p1_body(j, mxs):
        rb = row0 + j * G1
        pltpu.sync_copy(sidx2.at[pl.ds(rb, G1)], sidx_v.at[pl.ds(0, G1)])
        pltpu.sync_copy(ridx2.at[pl.ds(rb, G1)], ridx_v.at[pl.ds(0, G1)])
        pltpu.sync_copy(av_h.at[:, pl.ds(rb, G1), :], av)
        pltpu.sync_copy(bv_h.at[:, pl.ds(rb, G1), :], bv)
        hs = []
        for g in range(G1):
            hs.append(pltpu.async_copy(es_h.at[sidx_v.at[g]],
                                       esS.at[pl.ds(g * 128, 128)], sem))
            hs.append(pltpu.async_copy(es_h.at[ridx_v.at[g]],
                                       esR.at[pl.ds(g * 128, 128)], sem))
        for hh in hs:
            hh.wait()
        def p1_row(r2, mxs_in):
            new_mxs = list(mxs_in)
            for kk in range(H):
                kglob = c * H + kk
                cols = jnp.full((L,), kglob, jnp.int32)
                m = new_mxs[kk]
                for sub in range(128 // L):
                    rows = _iota16() + (r2 * 128 + sub * L)
                    vs = plsc.load_gather(esS, [rows, cols])
                    vr = plsc.load_gather(esR, [rows, cols])
                    d = vs - vr
                    off = sub * L
                    a = av[kk, r2, off:off + L]
                    b = bv[kk, r2, off:off + L]
                    lval = jnp.where(is_sc0, d * a + b, d)
                    lb[kk, r2, off:off + L] = lval
                    m = jnp.maximum(m, lval)
                new_mxs[kk] = m
            return tuple(new_mxs)

        new_mxs = lax.fori_loop(0, G1, p1_row, mxs)
        pltpu.sync_copy(lb.at[:, pl.ds(0, G1), :],
                        lscr.at[c, :, pl.ds(rb, G1), :])
        return new_mxs

    mxs0 = tuple(jnp.full((L,), -3.0e38, jnp.float32) for _ in range(H))
    mxs = lax.fori_loop(0, RPW // G1, p1_body, mxs0)
    for kk in range(H):
        mxv[kk] = mxs[kk]
    pltpu.sync_copy(mxv, maxsh.at[s])
    plsc.subcore_barrier()

    pltpu.sync_copy(maxsh, maxrd)
    M = []
    for kk in range(H):
        acc = maxrd[0, kk]
        for i in range(1, NS):
            acc = jnp.maximum(acc, maxrd[i, kk])
        M.append(jnp.max(acc))

    def p2_body(j, carry):
        rb = row0 + j * G2
        pltpu.sync_copy(ridx2.at[pl.ds(rb, G2)], ridx_v.at[pl.ds(0, G2)])
        pltpu.sync_copy(lscr.at[c, :, pl.ds(rb, G2), :],
                        lb.at[:, pl.ds(0, G2), :])

        def p2_row(r2, cc):
            for kk in range(H):
                cols = jnp.full((L,), kk, jnp.int32)
                for sub in range(128 // L):
                    off = sub * L
                    ex = jnp.exp(lb[kk, r2, off:off + L] - M[kk])
                    plsc.store_scatter(
                        exb, [_iota16() + (r2 * 128 + off), cols], ex)
            pltpu.sync_copy(exb.at[pl.ds(r2 * 128, 128)],
                            den_sh.at[ridx_v.at[r2]], add=True)
            return cc

        lax.fori_loop(0, G2, p2_row, 0)
        return carry

    lax.fori_loop(0, RPW // G2, p2_body, 0)
    plsc.subcore_barrier()
    pltpu.sync_copy(den_sh.at[pl.ds(nr0, NRW)],
                    denp.at[pl.dslice(c * NT + nr0, NRW)])
    plsc.subcore_barrier()

    def p3_pass(pf):
        def p3_batch(bi, carry):
            rb0 = row0 + bi * KB
            pltpu.sync_copy(sidx2.at[pl.ds(rb0, KB)], sidx_v)
            pltpu.sync_copy(ridx2.at[pl.ds(rb0, KB)], ridx_v)
            pltpu.sync_copy(lscr.at[c, :, pl.ds(rb0, KB), :], lb)

            def p3_row(r, cc):
                for g in range(128 // L):
                    vr = ridx_v[r, g * L:(g + 1) * L]
                    didx[g * L:(g + 1) * L] = vr + c * NT
                for kk in range(H):
                    koff = (c * H + kk) * N
                    for g in range(128 // L):
                        v = sidx_v[r, g * L:(g + 1) * L]
                        gidx[kk, g * L:(g + 1) * L] = v + koff
                hs = [pltpu.async_copy(denp.at[didx], db, sem)]
                hs += [pltpu.async_copy(pf.at[gidx.at[kk]], pg.at[kk], sem)
                       for kk in range(H)]
                for kk in range(H):
                    cols = jnp.full((L,), kk, jnp.int32)
                    for g in range(128 // L):
                        lv = lb[kk, r, g * L:(g + 1) * L]
                        ex = jnp.exp(lv - M[kk])
                        alb[kk, g * L:(g + 1) * L] = ex
                for hh in hs:
                    hh.wait()
                for kk in range(H):
                    cols = jnp.full((L,), kk, jnp.int32)
                    for g in range(128 // L):
                        dv = plsc.load_gather(db, [_iota16() + g * L, cols])
                        av_ = alb[kk, g * L:(g + 1) * L]
                        alb[kk, g * L:(g + 1) * L] = av_ / dv

                def grp_body(gi, cc2):
                    gsl = pl.ds(gi * L, L)
                    va = [alb[kk, gsl] for kk in range(H)]
                    for li in range(L):
                        row = gi * L + li
                        a0 = va[0][li]
                        a1 = va[1][li]
                        a2 = va[2][li]
                        a3 = va[3][li]
                        for fb in range(FH // L):
                            sl = pl.ds(fb * L, L)
                            acc = (a0 * pg[0, row, sl] + a1 * pg[1, row, sl]
                                   + a2 * pg[2, row, sl] + a3 * pg[3, row, sl])
                            combo[row, sl] = acc
                    return cc2

                lax.fori_loop(0, 128 // L, grp_body, 0)
                pltpu.sync_copy(combo, msg_sh.at[ridx_v.at[r]], add=True)
                return cc

            lax.fori_loop(0, KB, p3_row, 0)
            return carry

        lax.fori_loop(0, RPW // KB, p3_batch, 0)

    p3_pass(pf0)
    plsc.subcore_barrier()
    pltpu.sync_copy(msg_sh.at[pl.ds(nr0, NRW)], msgp.at[c, 0, pl.ds(nr0, NRW)])
    pltpu.sync_copy(zmsg.at[pl.ds(nr0, NRW)], msg_sh.at[pl.ds(nr0, NRW)])
    plsc.subcore_barrier()
    p3_pass(pf1)
    plsc.subcore_barrier()

    pltpu.sync_copy(msg_sh.at[pl.ds(nr0, NRW)], msgp.at[c, 1, pl.ds(nr0, NRW)])


def _k234(es_p, sidx2, ridx2, a2, b2, pf0, pf1, zmsg, zden):
    mesh = plsc.VectorSubcoreMesh(core_axis_name="c", subcore_axis_name="s")
    fn = functools.partial(
        pl.kernel,
        out_type=[
            jax.ShapeDtypeStruct((NC, 2, NT, FH), jnp.float32),
            jax.ShapeDtypeStruct((NC * NT, DW), jnp.float32),
            jax.ShapeDtypeStruct((NC, H, EROWS, 128), jnp.float32),
        ],
        mesh=mesh,
        compiler_params=pltpu.CompilerParams(use_tc_tiling_on_sc=False,
                                             needs_layout_passes=False),
        scratch_types=[
            pltpu.VMEM((KB, 128), jnp.int32),
            pltpu.VMEM((KB, 128), jnp.int32),
            pltpu.VMEM((G1 * 128, K2), jnp.float32),
            pltpu.VMEM((G1 * 128, K2), jnp.float32),
            pltpu.VMEM((H, G1, 128), jnp.float32),
            pltpu.VMEM((H, G1, 128), jnp.float32),
            pltpu.VMEM((H, KB, 128), jnp.float32),
            pltpu.VMEM((G2 * 128, DW), jnp.float32),
            pltpu.VMEM((H, L), jnp.float32),
            pltpu.VMEM((NS, H, L), jnp.float32),
            pltpu.VMEM((H, 128), jnp.int32),
            pltpu.VMEM((128,), jnp.int32),
            pltpu.VMEM((H, 128, FH), jnp.float32),
            pltpu.VMEM((128, FH), jnp.float32),
            pltpu.VMEM((128, DW), jnp.float32),
            pltpu.VMEM((H, 128), jnp.float32),
            pltpu.SemaphoreType.DMA,
            pltpu.VMEM_SHARED((NT, FH), jnp.float32),
            pltpu.VMEM_SHARED((NT, DW), jnp.float32),
            pltpu.VMEM_SHARED((NS, H, L), jnp.float32),
        ],
    )
    return fn(_k234_body)(es_p, sidx2, ridx2, a2, b2, pf0, pf1, zmsg, zden)


def _k5_body(x_ref, msg_ref, den_ref, p_ref, wout_ref, o_ref):
    xb = x_ref[...]
    acc = jnp.concatenate(
        [msg_ref[0, 0] + msg_ref[1, 0], msg_ref[0, 1] + msg_ref[1, 1]],
        axis=-1)
    for k in range(K2):
        dcol = den_ref[k // H][:, k % H]
        ind = jnp.where(dcol > 0.0, 1.0, 0.0)
        acc = acc - ind[:, None] * p_ref[k]
    acc = acc * (1.0 / H)
    acc = jnp.nan_to_num(acc)
    o_ref[...] = xb + jnp.dot(acc, wout_ref[...])


def _k5(x, msgp, denp, p, w_out):
    grid = N // _NB
    return pl.pallas_call(
        _k5_body,
        grid=(grid,),
        in_specs=[
            pl.BlockSpec((_NB, F), lambda i: (i, 0)),
            pl.BlockSpec((NC, 2, _NB, FH), lambda i: (0, 0, i, 0)),
            pl.BlockSpec((NC, _NB, DW), lambda i: (0, i, 0)),
            pl.BlockSpec((K2, _NB, F), lambda i: (0, i, 0)),
            pl.BlockSpec((F, F), lambda i: (0, 0)),
        ],
        out_specs=pl.BlockSpec((_NB, F), lambda i: (i, 0)),
        out_shape=jax.ShapeDtypeStruct((N, F), jnp.float32),
    )(x, msgp, denp, p, w_out)


def kernel(x, edge_index, edge_vec, edge_len, w_proj, radial_w, tangential_w,
           w_out, radial_score, tangential_score, log_scale, temp_bias,
           temp_weight):
    w_stack = jnp.concatenate([radial_w, tangential_w], axis=0)
    p, es = _k1(x, w_stack, w_proj, radial_score, tangential_score)
    a_e, b_e = _k1b(edge_len, temp_bias, temp_weight, log_scale)

    es_p = jnp.pad(es, ((0, NT - N), (0, 0)))
    sidx2 = jnp.pad(edge_index[0], (0, EP - E)).reshape(EROWS, 128)
    ridx2 = jnp.pad(edge_index[1], (0, EP - E),
                    constant_values=N).reshape(EROWS, 128)
    a2 = jnp.pad(a_e.reshape(H, E), ((0, 0), (0, EP - E))).reshape(
        H, EROWS, 128)
    b2 = jnp.pad(b_e.reshape(H, E), ((0, 0), (0, EP - E))).reshape(
        H, EROWS, 128)
    pflat = p.reshape(K2 * N, F)
    pf0 = pflat[:, :FH]
    pf1 = pflat[:, FH:]
    zmsg = jnp.zeros((NT, FH), jnp.float32)
    zden = jnp.zeros((NT, DW), jnp.float32)

    msgp, denp, _ = _k234(es_p, sidx2, ridx2, a2, b2, pf0, pf1, zmsg, zden)
    return _k5(x, msgp, denp.reshape(NC, NT, DW), p, w_out)


# speedup vs baseline: 24.8435x; 1.2859x over previous
import functools
import jax
import jax.numpy as jnp
from jax import lax
from jax.experimental import pallas as pl
from jax.experimental.pallas import tpu as pltpu
from jax.experimental.pallas import tpu_sc as plsc

N = 10000
F = 128
H = 4
K2 = 2 * H
E = 320000
L = 16
NC = 2
NS = 16
EP = 327680
EROWS = EP // 128
RPW = EROWS // NS
NT = 10112
NRW = NT // NS
G1 = 4
G2 = 5
KB = 8

_NB = 1000


def _k1_body(x_ref, w_ref, wproj_ref, rs_ref, ts_ref, p_ref, es_ref):
    xb = x_ref[...]
    cols = []
    for i in range(H):
        cols.append(jnp.dot(wproj_ref[i], rs_ref[i][:, None]))
    for i in range(H):
        cols.append(jnp.dot(wproj_ref[i], ts_ref[i][:, None]))
    vmat = jnp.concatenate(cols, axis=1)
    es_ref[...] = jnp.dot(xb, vmat)
    for k in range(K2):
        p_ref[k] = jnp.dot(xb, w_ref[k])


def _k1(x, w_stack, w_proj, r_score, t_score):
    grid = N // _NB
    return pl.pallas_call(
        _k1_body,
        grid=(grid,),
        in_specs=[
            pl.BlockSpec((_NB, F), lambda i: (i, 0)),
            pl.BlockSpec((K2, F, F), lambda i: (0, 0, 0)),
            pl.BlockSpec((H, F, F), lambda i: (0, 0, 0)),
            pl.BlockSpec((H, F), lambda i: (0, 0)),
            pl.BlockSpec((H, F), lambda i: (0, 0)),
        ],
        out_specs=[
            pl.BlockSpec((K2, _NB, F), lambda i: (0, i, 0)),
            pl.BlockSpec((_NB, K2), lambda i: (i, 0)),
        ],
        out_shape=[
            jax.ShapeDtypeStruct((K2, N, F), jnp.float32),
            jax.ShapeDtypeStruct((N, K2), jnp.float32),
        ],
    )(x, w_stack, w_proj, r_score, t_score)


def _k1b_body(len_ref, params_ref, ls_ref, a_ref, b_ref):
    lb = len_ref[...]
    scale = jax.nn.softplus(ls_ref[0])
    for k in range(H):
        t = jax.nn.softplus(params_ref[0, k] + params_ref[1, k] * lb)
        a = 1.0 / (t + 1e-4)
        a_ref[k] = a
        b_ref[k] = -scale * lb * a


def _k1b(edge_len, temp_bias, temp_weight, log_scale):
    len2 = edge_len.reshape(E // 128, 128)
    params = jnp.stack([temp_bias, temp_weight])
    ls = log_scale.reshape(1)
    return pl.pallas_call(
        _k1b_body,
        grid=(1,),
        in_specs=[
            pl.BlockSpec((E // 128, 128), lambda i: (0, 0)),
            pl.BlockSpec(memory_space=pltpu.SMEM),
            pl.BlockSpec(memory_space=pltpu.SMEM),
        ],
        out_specs=[
            pl.BlockSpec((H, E // 128, 128), lambda i: (0, 0, 0)),
            pl.BlockSpec((H, E // 128, 128), lambda i: (0, 0, 0)),
        ],
        out_shape=[
            jax.ShapeDtypeStruct((H, E // 128, 128), jnp.float32),
            jax.ShapeDtypeStruct((H, E // 128, 128), jnp.float32),
        ],
    )(len2, params, ls)


def _iota16():
    return lax.iota(jnp.int32, L)


FH = F // 2
DW = 8


def _k234_body(es_h, sidx2, ridx2, av_h, bv_h, pf0, pf1, zmsg, zden,
               msgp, denp, lscr,
               sidx_v, ridx_v, esS, esR, av, bv, lb, exb, mxv, maxrd,
               gidx, didx, pg, combo, db, alb, sem, semA, semB,
               msg_sh, den_sh, maxsh):
    c = lax.axis_index("c")
    s = lax.axis_index("s")
    is_sc0 = c == 0
    row0 = s * RPW
    nr0 = s * NRW

    pltpu.sync_copy(zmsg.at[pl.ds(nr0, NRW)], msg_sh.at[pl.ds(nr0, NRW)])
    pltpu.sync_copy(zden.at[pl.ds(nr0, NRW)], den_sh.at[pl.ds(nr0, NRW)])

    def p1_body(j, mxs):
        rb = row0 + j * G1
        pltpu.sync_copy(sidx2.at[pl.ds(rb, G1)], sidx_v.at[pl.ds(0, G1)])
        pltpu.sync_copy(ridx2.at[pl.ds(rb, G1)], ridx_v.at[pl.ds(0, G1)])
        pltpu.sync_copy(av_h.at[:, pl.ds(rb, G1), :], av)
        pltpu.sync_copy(bv_h.at[:, pl.ds(rb, G1), :], bv)
        hs = []
        for g in range(G1):
            hs.append(pltpu.async_copy(es_h.at[sidx_v.at[g]],
                                       esS.at[pl.ds(g * 128, 128)], sem))
            hs.append(pltpu.async_copy(es_h.at[ridx_v.at[g]],
                                       esR.at[pl.ds(g * 128, 128)], sem))
        for hh in hs:
            hh.wait()
        def p1_row(r2, mxs_in):
            new_mxs = list(mxs_in)
            for kk in range(H):
                kglob = c * H + kk
                cols = jnp.full((L,), kglob, jnp.int32)
                m = new_mxs[kk]
                for sub in range(128 // L):
                    rows = _iota16() + (r2 * 128 + sub * L)
                    vs = plsc.load_gather(esS, [rows, cols])
                    vr = plsc.load_gather(esR, [rows, cols])
                    d = vs - vr
                    off = sub * L
                    a = av[kk, r2, off:off + L]
                    b = bv[kk, r2, off:off + L]
                    lval = jnp.where(is_sc0, d * a + b, d)
                    lb[kk, r2, off:off + L] = lval
                    m = jnp.maximum(m, lval)
                new_mxs[kk] = m
            return tuple(new_mxs)

        new_mxs = lax.fori_loop(0, G1, p1_row, mxs)
        pltpu.sync_copy(lb.at[:, pl.ds(0, G1), :],
                        lscr.at[c, :, pl.ds(rb, G1), :])
        return new_mxs

    mxs0 = tuple(jnp.full((L,), -3.0e38, jnp.float32) for _ in range(H))
    mxs = lax.fori_loop(0, RPW // G1, p1_body, mxs0)
    for kk in range(H):
        mxv[kk] = mxs[kk]
    pltpu.sync_copy(mxv, maxsh.at[s])
    plsc.subcore_barrier()

    pltpu.sync_copy(maxsh, maxrd)
    M = []
    for kk in range(H):
        acc = maxrd[0, kk]
        for i in range(1, NS):
            acc = jnp.maximum(acc, maxrd[i, kk])
        M.append(jnp.max(acc))

    def p2_body(j, carry):
        rb = row0 + j * G2
        pltpu.sync_copy(ridx2.at[pl.ds(rb, G2)], ridx_v.at[pl.ds(0, G2)])
        pltpu.sync_copy(lscr.at[c, :, pl.ds(rb, G2), :],
                        lb.at[:, pl.ds(0, G2), :])

        def p2_row(r2, cc):
            for kk in range(H):
                cols = jnp.full((L,), kk, jnp.int32)
                for sub in range(128 // L):
                    off = sub * L
                    ex = jnp.exp(lb[kk, r2, off:off + L] - M[kk])
                    plsc.store_scatter(
                        exb, [_iota16() + (r2 * 128 + off), cols], ex)
            pltpu.sync_copy(exb.at[pl.ds(r2 * 128, 128)],
                            den_sh.at[ridx_v.at[r2]], add=True)
            return cc

        lax.fori_loop(0, G2, p2_row, 0)
        return carry

    lax.fori_loop(0, RPW // G2, p2_body, 0)
    plsc.subcore_barrier()
    pltpu.sync_copy(den_sh.at[pl.ds(nr0, NRW)],
                    denp.at[pl.dslice(c * NT + nr0, NRW)])
    plsc.subcore_barrier()

    def p3_pass(pf):
        sems = (semA, semB)

        def fire(r, b):
            for g in range(128 // L):
                vr = ridx_v[r, g * L:(g + 1) * L]
                didx[b, g * L:(g + 1) * L] = vr + c * NT
            for kk in range(H):
                koff = (c * H + kk) * N
                for g in range(128 // L):
                    v = sidx_v[r, g * L:(g + 1) * L]
                    gidx[b, kk, g * L:(g + 1) * L] = v + koff
            pltpu.async_copy(denp.at[didx.at[b]], db.at[b], sems[b])
            for kk in range(H):
                pltpu.async_copy(pf.at[gidx.at[b, kk]], pg.at[b, kk], sems[b])

        def drain(b):
            pltpu.make_async_copy(denp.at[didx.at[b]], db.at[b],
                                  sems[b]).wait()
            for kk in range(H):
                pltpu.make_async_copy(pf.at[gidx.at[b, kk]], pg.at[b, kk],
                                      sems[b]).wait()

        def compute_row(r, b):
            for kk in range(H):
                cols = jnp.full((L,), kk, jnp.int32)
                bcol = jnp.full((L,), b, jnp.int32)
                for g in range(128 // L):
                    lv = lb[kk, r, g * L:(g + 1) * L]
                    ex = jnp.exp(lv - M[kk])
                    dv = plsc.load_gather(db, [bcol, _iota16() + g * L, cols])
                    alb[kk, g * L:(g + 1) * L] = ex / dv

            def grp_body(gi, cc2):
                gsl = pl.ds(gi * L, L)
                va = [alb[kk, gsl] for kk in range(H)]
                for li in range(L):
                    row = gi * L + li
                    a0 = va[0][li]
                    a1 = va[1][li]
                    a2 = va[2][li]
                    a3 = va[3][li]
                    for j2 in range(FH // 32):
                        sl = pl.ds(j2 * 32, 32)
                        u0, v0 = plsc.unpack(pg[b, 0, row, sl],
                                             format=plsc.PackFormat.INTERLEAVED)
                        u1, v1 = plsc.unpack(pg[b, 1, row, sl],
                                             format=plsc.PackFormat.INTERLEAVED)
                        u2, v2 = plsc.unpack(pg[b, 2, row, sl],
                                             format=plsc.PackFormat.INTERLEAVED)
                        u3, v3 = plsc.unpack(pg[b, 3, row, sl],
                                             format=plsc.PackFormat.INTERLEAVED)
                        accu = a0 * u0 + a1 * u1 + a2 * u2 + a3 * u3
                        accv = a0 * v0 + a1 * v1 + a2 * v2 + a3 * v3
                        combo[row, pl.ds(j2 * 32, L)] = accu
                        combo[row, pl.ds(j2 * 32 + L, L)] = accv
                return cc2

            lax.fori_loop(0, 128 // L, grp_body, 0)
            pltpu.sync_copy(combo, msg_sh.at[ridx_v.at[r]], add=True)

        def p3_batch(bi, carry):
            rb0 = row0 + bi * KB
            pltpu.sync_copy(sidx2.at[pl.ds(rb0, KB)], sidx_v)
            pltpu.sync_copy(ridx2.at[pl.ds(rb0, KB)], ridx_v)
            pltpu.sync_copy(lscr.at[c, :, pl.ds(rb0, KB), :], lb)
            fire(0, 0)

            def p3_pair(jp, cc):
                for b in range(2):
                    r = 2 * jp + b

                    @pl.when(r + 1 < KB)
                    def _():
                        fire(r + 1, 1 - b)

                    drain(b)
                    compute_row(r, b)
                return cc

            lax.fori_loop(0, KB // 2, p3_pair, 0)
            return carry

        lax.fori_loop(0, RPW // KB, p3_batch, 0)

    p3_pass(pf0)
    plsc.subcore_barrier()
    pltpu.sync_copy(msg_sh.at[pl.ds(nr0, NRW)], msgp.at[c, 0, pl.ds(nr0, NRW)])
    pltpu.sync_copy(zmsg.at[pl.ds(nr0, NRW)], msg_sh.at[pl.ds(nr0, NRW)])
    plsc.subcore_barrier()
    p3_pass(pf1)
    plsc.subcore_barrier()

    pltpu.sync_copy(msg_sh.at[pl.ds(nr0, NRW)], msgp.at[c, 1, pl.ds(nr0, NRW)])


def _k234(es_p, sidx2, ridx2, a2, b2, pf0, pf1, zmsg, zden):
    mesh = plsc.VectorSubcoreMesh(core_axis_name="c", subcore_axis_name="s")
    fn = functools.partial(
        pl.kernel,
        out_type=[
            jax.ShapeDtypeStruct((NC, 2, NT, FH), jnp.float32),
            jax.ShapeDtypeStruct((NC * NT, DW), jnp.float32),
            jax.ShapeDtypeStruct((NC, H, EROWS, 128), jnp.float32),
        ],
        mesh=mesh,
        compiler_params=pltpu.CompilerParams(use_tc_tiling_on_sc=False,
                                             needs_layout_passes=False),
        scratch_types=[
            pltpu.VMEM((KB, 128), jnp.int32),
            pltpu.VMEM((KB, 128), jnp.int32),
            pltpu.VMEM((G1 * 128, K2), jnp.float32),
            pltpu.VMEM((G1 * 128, K2), jnp.float32),
            pltpu.VMEM((H, G1, 128), jnp.float32),
            pltpu.VMEM((H, G1, 128), jnp.float32),
            pltpu.VMEM((H, KB, 128), jnp.float32),
            pltpu.VMEM((G2 * 128, DW), jnp.float32),
            pltpu.VMEM((H, L), jnp.float32),
            pltpu.VMEM((NS, H, L), jnp.float32),
            pltpu.VMEM((2, H, 128), jnp.int32),
            pltpu.VMEM((2, 128), jnp.int32),
            pltpu.VMEM((2, H, 128, FH), jnp.bfloat16),
            pltpu.VMEM((128, FH), jnp.float32),
            pltpu.VMEM((2, 128, DW), jnp.float32),
            pltpu.VMEM((H, 128), jnp.float32),
            pltpu.SemaphoreType.DMA,
            pltpu.SemaphoreType.DMA,
            pltpu.SemaphoreType.DMA,
            pltpu.VMEM_SHARED((NT, FH), jnp.float32),
            pltpu.VMEM_SHARED((NT, DW), jnp.float32),
            pltpu.VMEM_SHARED((NS, H, L), jnp.float32),
        ],
    )
    return fn(_k234_body)(es_p, sidx2, ridx2, a2, b2, pf0, pf1, zmsg, zden)


def _k5_body(x_ref, msg_ref, den_ref, p_ref, wout_ref, o_ref):
    xb = x_ref[...]
    acc = jnp.concatenate(
        [msg_ref[0, 0] + msg_ref[1, 0], msg_ref[0, 1] + msg_ref[1, 1]],
        axis=-1)
    for k in range(K2):
        dcol = den_ref[k // H][:, k % H]
        ind = jnp.where(dcol > 0.0, 1.0, 0.0)
        acc = acc - ind[:, None] * p_ref[k]
    acc = acc * (1.0 / H)
    acc = jnp.nan_to_num(acc)
    o_ref[...] = xb + jnp.dot(acc, wout_ref[...])


def _k5(x, msgp, denp, p, w_out):
    grid = N // _NB
    return pl.pallas_call(
        _k5_body,
        grid=(grid,),
        in_specs=[
            pl.BlockSpec((_NB, F), lambda i: (i, 0)),
            pl.BlockSpec((NC, 2, _NB, FH), lambda i: (0, 0, i, 0)),
            pl.BlockSpec((NC, _NB, DW), lambda i: (0, i, 0)),
            pl.BlockSpec((K2, _NB, F), lambda i: (0, i, 0)),
            pl.BlockSpec((F, F), lambda i: (0, 0)),
        ],
        out_specs=pl.BlockSpec((_NB, F), lambda i: (i, 0)),
        out_shape=jax.ShapeDtypeStruct((N, F), jnp.float32),
    )(x, msgp, denp, p, w_out)


def kernel(x, edge_index, edge_vec, edge_len, w_proj, radial_w, tangential_w,
           w_out, radial_score, tangential_score, log_scale, temp_bias,
           temp_weight):
    w_stack = jnp.concatenate([radial_w, tangential_w], axis=0)
    p, es = _k1(x, w_stack, w_proj, radial_score, tangential_score)
    a_e, b_e = _k1b(edge_len, temp_bias, temp_weight, log_scale)

    es_p = jnp.pad(es, ((0, NT - N), (0, 0)))
    sidx2 = jnp.pad(edge_index[0], (0, EP - E)).reshape(EROWS, 128)
    ridx2 = jnp.pad(edge_index[1], (0, EP - E),
                    constant_values=N).reshape(EROWS, 128)
    a2 = jnp.pad(a_e.reshape(H, E), ((0, 0), (0, EP - E))).reshape(
        H, EROWS, 128)
    b2 = jnp.pad(b_e.reshape(H, E), ((0, 0), (0, EP - E))).reshape(
        H, EROWS, 128)
    pflat = p.reshape(K2 * N, F)
    perm = []
    for pcol in range(FH):
        blk, q = pcol // 32, pcol % 32
        perm.append(32 * blk + (q // 2 if q % 2 == 0 else 16 + q // 2))
    perm = jnp.array(perm, jnp.int32)
    pf0 = pflat[:, :FH][:, perm].astype(jnp.bfloat16)
    pf1 = pflat[:, FH:][:, perm].astype(jnp.bfloat16)
    zmsg = jnp.zeros((NT, FH), jnp.float32)
    zden = jnp.zeros((NT, DW), jnp.float32)

    msgp, denp, _ = _k234(es_p, sidx2, ridx2, a2, b2, pf0, pf1, zmsg, zden)
    return _k5(x, msgp, denp.reshape(NC, NT, DW), p, w_out)
